# Initial kernel scaffold; baseline (speedup 1.0000x reference)
import jax
import jax.numpy as jnp
from jax.experimental import pallas as pl


def kernel(wmat1, bc1, wmat2, bc2, W1, B1, W2, B2, W3, B3, x):
    raise NotImplementedError("write your pallas kernel here")


# speedup vs baseline: 532.3351x; 532.3351x over previous
import jax
import jax.numpy as jnp
from jax.experimental import pallas as pl
from jax.experimental.pallas import tpu as pltpu

_NB = 128


def _lenet_kernel(x0_ref, x1_ref, x2_ref, x3_ref, t1_ref, b1_ref, t2_ref,
                  b2_ref, f1_ref, fb1_ref, w2_ref, fb2_ref, w3_ref, fb3_ref,
                  o_ref):
    nb = x0_ref.shape[0]
    xm = [x0_ref[...].reshape(nb * 8, 96), x1_ref[...].reshape(nb * 8, 96),
          x2_ref[...].reshape(nb * 8, 96), x3_ref[...].reshape(nb * 8, 96)]

    b1 = b1_ref[...].reshape(1, 1, 336)
    racc = [[None, None], [None, None]]
    for e2 in range(2):
        for e1 in range(2):
            for i in range(5):
                q = 2 * e2 + e1 + i
                z = jnp.dot(xm[q % 4], t1_ref[i],
                            preferred_element_type=jnp.float32)
                z = z.reshape(nb, 8, 336)[:, q // 4:q // 4 + 7, :]
                racc[e2][e1] = z if racc[e2][e1] is None else racc[e2][e1] + z
    sm = [jnp.maximum(racc[e2][0] + b1, 0.0)
          + jnp.maximum(racc[e2][1] + b1, 0.0) for e2 in range(2)]
    sm = [s.reshape(nb * 7, 336) for s in sm]

    b2 = b2_ref[...].reshape(1, 1, 320)
    acc2 = [None, None]
    for e2b in range(2):
        for k in range(5):
            q = e2b + k
            z = jnp.dot(sm[q % 2], t2_ref[k],
                        preferred_element_type=jnp.float32)
            z = z.reshape(nb, 7, 320)[:, q // 2:q // 2 + 5, :]
            acc2[e2b] = z if acc2[e2b] is None else acc2[e2b] + z
    s2 = jnp.maximum(acc2[0] + b2, 0.0) + jnp.maximum(acc2[1] + b2, 0.0)

    hh = None
    for rr in range(5):
        z = jnp.dot(s2[:, rr, :], f1_ref[rr], preferred_element_type=jnp.float32)
        hh = z if hh is None else hh + z
    h1 = jnp.maximum(hh + fb1_ref[...], 0.0)
    h2 = jnp.maximum(
        jnp.dot(h1, w2_ref[...], preferred_element_type=jnp.float32)
        + fb2_ref[...], 0.0)
    logits = jnp.dot(h2, w3_ref[...], preferred_element_type=jnp.float32) \
        + fb3_ref[...]
    col = jax.lax.broadcasted_iota(jnp.int32, logits.shape, 1)
    logits = jnp.where(col < 10, logits, jnp.float32(-1e30))
    m = jnp.max(logits, axis=-1, keepdims=True)
    e = jnp.exp(logits - m)
    o_ref[...] = e / jnp.sum(e, axis=-1, keepdims=True)


def _build_tables(wmat1, bc1, wmat2, bc2, W1):
    i_ = jnp.arange(5)[:, None, None]
    wi = (jnp.arange(96) // 3)[None, :, None]
    c = (jnp.arange(96) % 3)[None, :, None]
    wo = jnp.arange(28)[None, None, :]
    j = wi - wo
    ridx = i_ * 15 + j * 3 + c
    valid = (j >= 0) & (j < 5)
    t1 = jnp.where(valid[..., None],
                   wmat1[jnp.clip(ridx, 0, wmat1.shape[0] - 1)][..., :12], 0.0)
    t1 = t1.reshape(5, 96, 336)

    k_ = jnp.arange(5)[:, None, None]
    wo1 = (jnp.arange(336) // 12)[None, :, None]
    c1 = (jnp.arange(336) % 12)[None, :, None]
    wo2 = jnp.arange(10)[None, None, :]
    j2 = wo1 // 2 - wo2
    ridx2 = k_ * 80 + j2 * 16 + c1
    valid2 = (j2 >= 0) & (j2 < 5)
    t2 = jnp.where(valid2[..., None],
                   wmat2[jnp.clip(ridx2, 0, wmat2.shape[0] - 1)][..., :32], 0.0)
    t2 = (0.25 * t2).reshape(5, 336, 320)

    r_ = jnp.arange(5)[:, None]
    lane = jnp.arange(320)[None, :]
    ridx3 = (r_ * 5 + (lane // 64)) * 32 + (lane % 32)
    f1 = 0.25 * W1[ridx3]

    b1v = jnp.tile(bc1[0, :12], 28).reshape(1, 336)
    b2v = jnp.tile(bc2[0, :32], 10).reshape(1, 320)
    return t1, b1v, t2, b2v, f1


def kernel(wmat1, bc1, wmat2, bc2, W1, B1, W2, B2, W3, B3, x):
    N = x.shape[0]
    x3 = jnp.transpose(x, (0, 2, 3, 1)).reshape(N, 32, 96)
    t1, b1v, t2, b2v, f1 = _build_tables(wmat1, bc1, wmat2, bc2, W1)

    nb = _NB
    Np = (N + nb - 1) // nb * nb
    if Np != N:
        x3 = jnp.pad(x3, ((0, Np - N), (0, 0), (0, 0)))
    xs = [x3[:, m::4, :] for m in range(4)]
    full = lambda shape: pl.BlockSpec(shape, lambda m: tuple(0 for _ in shape))
    out = pl.pallas_call(
        _lenet_kernel,
        out_shape=jax.ShapeDtypeStruct((Np, 128), jnp.float32),
        grid=(Np // nb,),
        in_specs=[
            pl.BlockSpec((nb, 8, 96), lambda m: (m, 0, 0)),
            pl.BlockSpec((nb, 8, 96), lambda m: (m, 0, 0)),
            pl.BlockSpec((nb, 8, 96), lambda m: (m, 0, 0)),
            pl.BlockSpec((nb, 8, 96), lambda m: (m, 0, 0)),
            full((5, 96, 336)), full((1, 336)),
            full((5, 336, 320)), full((1, 320)),
            full((5, 320, 128)), full((1, 128)),
            full((128, 128)), full((1, 128)),
            full((128, 128)), full((1, 128)),
        ],
        out_specs=pl.BlockSpec((nb, 128), lambda m: (m, 0)),
        compiler_params=pltpu.CompilerParams(
            dimension_semantics=("parallel",),
            vmem_limit_bytes=100 * 1024 * 1024),
        cost_estimate=pl.CostEstimate(
            flops=2 * Np * (5 * 32 * 96 * 336 + 5 * 14 * 336 * 320
                            + 5 * 320 * 128 + 128 * 128 * 2),
            transcendentals=Np * 128,
            bytes_accessed=(Np * 32 * 96 + Np * 128) * 4),
    )(xs[0], xs[1], xs[2], xs[3], t1, b1v, t2, b2v, f1, B1, W2, B2, W3, B3)
    return out[:N, :10]
